# baseline (speedup 1.0000x reference)
import functools

import jax
import jax.numpy as jnp
from jax import lax
from jax.experimental import pallas as pl
from jax.experimental.pallas import tpu as pltpu
from jax.experimental.pallas import tpu_sc as plsc

N = 10000
E = 160000
D = 256
H = 128
NSC = 2
NT = 16
CH = 128
CHUNKS = 80
HALF = CHUNKS // 2
E_PAD = NT * CHUNKS * CH
N_PAD = 10112
ZROWS = N_PAD // NT

_SC_MESH = plsc.VectorSubcoreMesh(core_axis_name="c", subcore_axis_name="s")


@functools.partial(
    pl.kernel,
    out_type=jax.ShapeDtypeStruct((NSC, N_PAD, H), jnp.float32),
    mesh=_SC_MESH,
    scratch_types=[
        pltpu.VMEM((HALF, CH), jnp.int32),
        pltpu.VMEM((HALF, CH), jnp.int32),
        pltpu.VMEM((2, CH, H), jnp.float32),
        pltpu.VMEM_SHARED((N_PAD, H), jnp.float32),
        pltpu.SemaphoreType.DMA((2,)),
        pltpu.SemaphoreType.DMA((2,)),
    ],
)
def _sc_agg(tbl, src3, dst3, zeros, out, sidx, didx, rows, acc, sems, ssems):
    c = lax.axis_index("c")
    s = lax.axis_index("s")
    pltpu.sync_copy(zeros, acc.at[pl.ds(s * ZROWS, ZROWS)])
    plsc.subcore_barrier()

    for ph in (0, 1):
        cbase = s * CHUNKS + ph * HALF
        pltpu.sync_copy(src3.at[c, pl.ds(cbase, HALF)], sidx)
        pltpu.sync_copy(dst3.at[pl.ds(cbase, HALF)], didx)
        pltpu.async_copy(tbl.at[sidx.at[0]], rows.at[0], sems.at[0])

        @pl.loop(0, HALF, step=2)
        def _(g):
            for b in (0, 1):
                i = g + b
                nb = 1 - b

                pltpu.make_async_copy(tbl.at[sidx.at[i]], rows.at[b],
                                      sems.at[b]).wait()
                pltpu.async_copy(rows.at[b], acc.at[didx.at[i]],
                                 ssems.at[b], add=True)

                @pl.when(i >= 1)
                def _():
                    pltpu.make_async_copy(rows.at[nb],
                                          acc.at[didx.at[i - 1]],
                                          ssems.at[nb]).wait()

                @pl.when(i + 1 < HALF)
                def _():
                    pltpu.async_copy(tbl.at[sidx.at[i + 1]],
                                     rows.at[nb], sems.at[nb])

        pltpu.make_async_copy(rows.at[1], acc.at[didx.at[HALF - 1]],
                              ssems.at[1]).wait()

    plsc.subcore_barrier()
    pltpu.sync_copy(acc.at[pl.ds(s * ZROWS, ZROWS)],
                    out.at[c, pl.ds(s * ZROWS, ZROWS)])


_R = 400


def _tc_body(relu_after, xs_ref, agg_ref, wa_ref, ba_ref, wb_ref, bb_ref,
             out_ref):
    h = jnp.concatenate(
        [xs_ref[0] + agg_ref[0], xs_ref[1] + agg_ref[1]], axis=-1)
    h = jnp.dot(h, wa_ref[...], preferred_element_type=jnp.float32)
    h = jnp.maximum(h + ba_ref[...], 0.0)
    h = jnp.dot(h, wb_ref[...], preferred_element_type=jnp.float32)
    h = h + bb_ref[...]
    if relu_after:
        h = jnp.maximum(h, 0.0)
        out_ref[0] = h[:, :H]
        out_ref[1] = h[:, H:]
    else:
        out_ref[...] = h


def _tc_mlp_call(relu_after, xs, agg, wa, ba, wb, bb):
    grid = N // _R
    split_spec = pl.BlockSpec((NSC, _R, H), lambda i: (0, i, 0))
    full = pl.BlockSpec((D, D), lambda i: (0, 0))
    bias = pl.BlockSpec((1, D), lambda i: (0, 0))
    if relu_after:
        out_shape = jax.ShapeDtypeStruct((NSC, N, H), jnp.float32)
        out_spec = split_spec
    else:
        out_shape = jax.ShapeDtypeStruct((N, D), jnp.float32)
        out_spec = pl.BlockSpec((_R, D), lambda i: (i, 0))
    return pl.pallas_call(
        functools.partial(_tc_body, relu_after),
        grid=(grid,),
        in_specs=[split_spec, split_spec, full, bias, full, bias],
        out_specs=out_spec,
        out_shape=out_shape,
    )(xs, agg, wa, ba, wb, bb)


def kernel(x, edge_index, W1a, b1a, W1b, b1b, W2a, b2a, W2b, b2b):
    src = edge_index[0]
    dst = edge_index[1]
    pad = E_PAD - E
    srcp = jnp.concatenate([src, jnp.zeros((pad,), jnp.int32)])
    src3 = jnp.stack([srcp, srcp + N]).reshape(NSC, NT * CHUNKS, CH)
    dst3 = jnp.concatenate([dst, jnp.full((pad,), N, jnp.int32)])
    dst3 = dst3.reshape(NT * CHUNKS, CH)
    zeros = jnp.zeros((ZROWS, H), jnp.float32)

    x2 = jnp.stack([x[:, :H], x[:, H:]])
    agg1 = _sc_agg(x2.reshape(NSC * N, H), src3, dst3, zeros)
    h2 = _tc_mlp_call(True, x2, agg1, W1a, b1a.reshape(1, D), W1b,
                      b1b.reshape(1, D))
    agg2 = _sc_agg(h2.reshape(NSC * N, H), src3, dst3, zeros)
    out = _tc_mlp_call(False, h2, agg2, W2a, b2a.reshape(1, D), W2b,
                       b2b.reshape(1, D))
    return out


# speedup vs baseline: 1.0614x; 1.0614x over previous
import functools

import jax
import jax.numpy as jnp
from jax import lax
from jax.experimental import pallas as pl
from jax.experimental.pallas import tpu as pltpu
from jax.experimental.pallas import tpu_sc as plsc

N = 10000
E = 160000
D = 256
H = 128
NSC = 2
NT = 16
CH = 128
CHUNKS = 80
HALF = CHUNKS // 2
E_PAD = NT * CHUNKS * CH
N_PAD = 10112
ZROWS = N_PAD // NT

_SC_MESH = plsc.VectorSubcoreMesh(core_axis_name="c", subcore_axis_name="s")


@functools.partial(
    pl.kernel,
    out_type=jax.ShapeDtypeStruct((NSC, N_PAD, H), jnp.float32),
    mesh=_SC_MESH,
    scratch_types=[
        pltpu.VMEM((HALF, CH), jnp.int32),
        pltpu.VMEM((HALF, CH), jnp.int32),
        pltpu.VMEM((2, CH, H), jnp.float32),
        pltpu.VMEM_SHARED((N_PAD, H), jnp.float32),
        pltpu.SemaphoreType.DMA((2,)),
        pltpu.SemaphoreType.DMA((2,)),
    ],
)
def _sc_agg(tbl, src3, dst3, zeros, out, sidx, didx, rows, acc, sems, ssems):
    c = lax.axis_index("c")
    s = lax.axis_index("s")
    pltpu.sync_copy(zeros, acc.at[pl.ds(s * ZROWS, ZROWS)])
    plsc.subcore_barrier()

    for ph in (0, 1):
        cbase = s * CHUNKS + ph * HALF
        pltpu.sync_copy(src3.at[c, pl.ds(cbase, HALF)], sidx)
        pltpu.sync_copy(dst3.at[pl.ds(cbase, HALF)], didx)
        pltpu.async_copy(tbl.at[sidx.at[0]], rows.at[0], sems.at[0])

        @pl.loop(0, HALF, step=2)
        def _(g):
            for b in (0, 1):
                i = g + b
                nb = 1 - b

                @pl.when(i >= 1)
                def _():
                    pltpu.make_async_copy(rows.at[nb],
                                          acc.at[didx.at[i - 1]],
                                          ssems.at[nb]).wait()

                @pl.when(i + 1 < HALF)
                def _():
                    pltpu.async_copy(tbl.at[sidx.at[i + 1]],
                                     rows.at[nb], sems.at[nb])

                pltpu.make_async_copy(tbl.at[sidx.at[i]], rows.at[b],
                                      sems.at[b]).wait()
                pltpu.async_copy(rows.at[b], acc.at[didx.at[i]],
                                 ssems.at[b], add=True)

        pltpu.make_async_copy(rows.at[1], acc.at[didx.at[HALF - 1]],
                              ssems.at[1]).wait()

    plsc.subcore_barrier()
    pltpu.sync_copy(acc.at[pl.ds(s * ZROWS, ZROWS)],
                    out.at[c, pl.ds(s * ZROWS, ZROWS)])


_R = 400


def _tc_body(relu_after, xs_ref, agg_ref, wa_ref, ba_ref, wb_ref, bb_ref,
             out_ref):
    h = jnp.concatenate(
        [xs_ref[0] + agg_ref[0], xs_ref[1] + agg_ref[1]], axis=-1)
    h = jnp.dot(h, wa_ref[...], preferred_element_type=jnp.float32)
    h = jnp.maximum(h + ba_ref[...], 0.0)
    h = jnp.dot(h, wb_ref[...], preferred_element_type=jnp.float32)
    h = h + bb_ref[...]
    if relu_after:
        h = jnp.maximum(h, 0.0)
        out_ref[0] = h[:, :H]
        out_ref[1] = h[:, H:]
    else:
        out_ref[...] = h


def _tc_mlp_call(relu_after, xs, agg, wa, ba, wb, bb):
    grid = N // _R
    split_spec = pl.BlockSpec((NSC, _R, H), lambda i: (0, i, 0))
    full = pl.BlockSpec((D, D), lambda i: (0, 0))
    bias = pl.BlockSpec((1, D), lambda i: (0, 0))
    if relu_after:
        out_shape = jax.ShapeDtypeStruct((NSC, N, H), jnp.float32)
        out_spec = split_spec
    else:
        out_shape = jax.ShapeDtypeStruct((N, D), jnp.float32)
        out_spec = pl.BlockSpec((_R, D), lambda i: (i, 0))
    return pl.pallas_call(
        functools.partial(_tc_body, relu_after),
        grid=(grid,),
        in_specs=[split_spec, split_spec, full, bias, full, bias],
        out_specs=out_spec,
        out_shape=out_shape,
    )(xs, agg, wa, ba, wb, bb)


def kernel(x, edge_index, W1a, b1a, W1b, b1b, W2a, b2a, W2b, b2b):
    src = edge_index[0]
    dst = edge_index[1]
    pad = E_PAD - E
    srcp = jnp.concatenate([src, jnp.zeros((pad,), jnp.int32)])
    src3 = jnp.stack([srcp, srcp + N]).reshape(NSC, NT * CHUNKS, CH)
    dst3 = jnp.concatenate([dst, jnp.full((pad,), N, jnp.int32)])
    dst3 = dst3.reshape(NT * CHUNKS, CH)
    zeros = jnp.zeros((ZROWS, H), jnp.float32)

    x2 = jnp.stack([x[:, :H], x[:, H:]])
    agg1 = _sc_agg(x2.reshape(NSC * N, H), src3, dst3, zeros)
    h2 = _tc_mlp_call(True, x2, agg1, W1a, b1a.reshape(1, D), W1b,
                      b1b.reshape(1, D))
    agg2 = _sc_agg(h2.reshape(NSC * N, H), src3, dst3, zeros)
    out = _tc_mlp_call(False, h2, agg2, W2a, b2a.reshape(1, D), W2b,
                       b2b.reshape(1, D))
    return out


# speedup vs baseline: 1.1838x; 1.1153x over previous
import functools

import jax
import jax.numpy as jnp
from jax import lax
from jax.experimental import pallas as pl
from jax.experimental.pallas import tpu as pltpu
from jax.experimental.pallas import tpu_sc as plsc

N = 10000
E = 160000
D = 256
H = 128
NSC = 2
NT = 16
CH = 128
CHUNKS = 80
HALF = CHUNKS // 2
E_PAD = NT * CHUNKS * CH
N_PAD = 10240
ZROWS = N_PAD // NT

_SC_MESH = plsc.VectorSubcoreMesh(core_axis_name="c", subcore_axis_name="s")


@functools.partial(
    pl.kernel,
    out_type=jax.ShapeDtypeStruct((NSC, N_PAD, H), jnp.float32),
    mesh=_SC_MESH,
    scratch_types=[
        pltpu.VMEM((HALF, CH), jnp.int32),
        pltpu.VMEM((HALF, CH), jnp.int32),
        pltpu.VMEM((2, CH, H), jnp.float32),
        pltpu.VMEM_SHARED((N_PAD, H), jnp.float32),
        pltpu.SemaphoreType.DMA((2,)),
        pltpu.SemaphoreType.DMA((2,)),
    ],
)
def _sc_agg(tbl, src3, dst3, zeros, out, sidx, didx, rows, acc, sems, ssems):
    c = lax.axis_index("c")
    s = lax.axis_index("s")
    pltpu.sync_copy(zeros, acc.at[pl.ds(s * ZROWS, ZROWS)])
    plsc.subcore_barrier()

    for ph in (0, 1):
        cbase = s * CHUNKS + ph * HALF
        pltpu.sync_copy(src3.at[c, pl.ds(cbase, HALF)], sidx)
        pltpu.sync_copy(dst3.at[pl.ds(cbase, HALF)], didx)
        pltpu.async_copy(tbl.at[sidx.at[0]], rows.at[0], sems.at[0])

        @pl.loop(0, HALF, step=2)
        def _(g):
            for b in (0, 1):
                i = g + b
                nb = 1 - b

                @pl.when(i >= 1)
                def _():
                    pltpu.make_async_copy(rows.at[nb],
                                          acc.at[didx.at[i - 1]],
                                          ssems.at[nb]).wait()

                @pl.when(i + 1 < HALF)
                def _():
                    pltpu.async_copy(tbl.at[sidx.at[i + 1]],
                                     rows.at[nb], sems.at[nb])

                pltpu.make_async_copy(tbl.at[sidx.at[i]], rows.at[b],
                                      sems.at[b]).wait()
                pltpu.async_copy(rows.at[b], acc.at[didx.at[i]],
                                 ssems.at[b], add=True)

        pltpu.make_async_copy(rows.at[1], acc.at[didx.at[HALF - 1]],
                              ssems.at[1]).wait()

    plsc.subcore_barrier()
    pltpu.sync_copy(acc.at[pl.ds(s * ZROWS, ZROWS)],
                    out.at[c, pl.ds(s * ZROWS, ZROWS)])


_R = 400


def _tc_body(relu_after, xs_ref, agg_ref, wa_ref, ba_ref, wb_ref, bb_ref,
             out_ref):
    h = jnp.concatenate(
        [xs_ref[0] + agg_ref[0], xs_ref[1] + agg_ref[1]], axis=-1)
    h = jnp.dot(h, wa_ref[...], preferred_element_type=jnp.float32)
    h = jnp.maximum(h + ba_ref[...], 0.0)
    h = jnp.dot(h, wb_ref[...], preferred_element_type=jnp.float32)
    h = h + bb_ref[...]
    if relu_after:
        h = jnp.maximum(h, 0.0)
        out_ref[0] = h[:, :H]
        out_ref[1] = h[:, H:]
    else:
        out_ref[...] = h


def _tc_mlp_call(relu_after, xs, agg, wa, ba, wb, bb):
    grid = N // _R
    split_spec = pl.BlockSpec((NSC, _R, H), lambda i: (0, i, 0))
    full = pl.BlockSpec((D, D), lambda i: (0, 0))
    bias = pl.BlockSpec((1, D), lambda i: (0, 0))
    if relu_after:
        out_shape = jax.ShapeDtypeStruct((NSC, N, H), jnp.float32)
        out_spec = split_spec
    else:
        out_shape = jax.ShapeDtypeStruct((N, D), jnp.float32)
        out_spec = pl.BlockSpec((_R, D), lambda i: (i, 0))
    return pl.pallas_call(
        functools.partial(_tc_body, relu_after),
        grid=(grid,),
        in_specs=[split_spec, split_spec, full, bias, full, bias],
        out_specs=out_spec,
        out_shape=out_shape,
    )(xs, agg, wa, ba, wb, bb)


def kernel(x, edge_index, W1a, b1a, W1b, b1b, W2a, b2a, W2b, b2b):
    src = edge_index[0]
    dst = edge_index[1]
    pad = E_PAD - E
    srcp = jnp.concatenate([src, jnp.zeros((pad,), jnp.int32)])
    src3 = jnp.stack([srcp, srcp + N]).reshape(NSC, NT * CHUNKS, CH)
    dst3 = jnp.concatenate([dst, jnp.full((pad,), N, jnp.int32)])
    dst3 = dst3.reshape(NT * CHUNKS, CH)
    zeros = jnp.zeros((ZROWS, H), jnp.float32)

    x2 = jnp.stack([x[:, :H], x[:, H:]])
    agg1 = _sc_agg(x2.reshape(NSC * N, H), src3, dst3, zeros)
    h2 = _tc_mlp_call(True, x2, agg1, W1a, b1a.reshape(1, D), W1b,
                      b1b.reshape(1, D))
    agg2 = _sc_agg(h2.reshape(NSC * N, H), src3, dst3, zeros)
    out = _tc_mlp_call(False, h2, agg2, W2a, b2a.reshape(1, D), W2b,
                       b2b.reshape(1, D))
    return out


# speedup vs baseline: 1.2041x; 1.0172x over previous
import functools

import jax
import jax.numpy as jnp
from jax import lax
from jax.experimental import pallas as pl
from jax.experimental.pallas import tpu as pltpu
from jax.experimental.pallas import tpu_sc as plsc

N = 10000
E = 160000
D = 256
H = 128
NSC = 2
NT = 16
CH = 128
CHUNKS = 80
HALF = CHUNKS // 2
E_PAD = NT * CHUNKS * CH
N_PAD = 10240
ZROWS = N_PAD // NT

_SC_MESH = plsc.VectorSubcoreMesh(core_axis_name="c", subcore_axis_name="s")


@functools.partial(
    pl.kernel,
    out_type=jax.ShapeDtypeStruct((NSC, N_PAD, H), jnp.float32),
    mesh=_SC_MESH,
    scratch_types=[
        pltpu.VMEM((HALF, CH), jnp.int32),
        pltpu.VMEM((HALF, CH), jnp.int32),
        pltpu.VMEM((2, CH, H), jnp.float32),
        pltpu.VMEM_SHARED((N_PAD, H), jnp.float32),
        pltpu.SemaphoreType.DMA((2,)),
        pltpu.SemaphoreType.DMA((2,)),
    ],
)
def _sc_agg(tbl, src3, dst3, zeros, out, sidx, didx, rows, acc, sems, ssems):
    c = lax.axis_index("c")
    s = lax.axis_index("s")
    pltpu.sync_copy(zeros, acc.at[pl.ds(s * ZROWS, ZROWS)])
    plsc.subcore_barrier()

    for ph in (0, 1):
        cbase = s * CHUNKS + ph * HALF
        pltpu.sync_copy(src3.at[c, pl.ds(cbase, HALF)], sidx)
        pltpu.sync_copy(dst3.at[pl.ds(cbase, HALF)], didx)
        pltpu.async_copy(tbl.at[sidx.at[0]], rows.at[0], sems.at[0])

        @pl.loop(0, HALF, step=2)
        def _(g):
            for b in (0, 1):
                i = g + b
                nb = 1 - b

                @pl.when(i >= 1)
                def _():
                    pltpu.make_async_copy(rows.at[nb],
                                          acc.at[didx.at[i - 1]],
                                          ssems.at[nb]).wait()

                @pl.when(i + 1 < HALF)
                def _():
                    pltpu.async_copy(tbl.at[sidx.at[i + 1]],
                                     rows.at[nb], sems.at[nb])

                pltpu.make_async_copy(tbl.at[sidx.at[i]], rows.at[b],
                                      sems.at[b]).wait()
                pltpu.async_copy(rows.at[b], acc.at[didx.at[i]],
                                 ssems.at[b], add=True)

        pltpu.make_async_copy(rows.at[1], acc.at[didx.at[HALF - 1]],
                              ssems.at[1]).wait()

    plsc.subcore_barrier()
    pltpu.sync_copy(acc.at[pl.ds(s * ZROWS, ZROWS)],
                    out.at[c, pl.ds(s * ZROWS, ZROWS)])


_R = 400


def _tc_body(relu_after, xs_ref, agg_ref, wa_ref, ba_ref, wb_ref, bb_ref,
             out_ref):
    h = jnp.concatenate(
        [xs_ref[0] + agg_ref[0], xs_ref[1] + agg_ref[1]], axis=-1)
    h = jnp.dot(h.astype(jnp.bfloat16), wa_ref[...].astype(jnp.bfloat16),
                preferred_element_type=jnp.float32)
    h = jnp.maximum(h + ba_ref[...], 0.0)
    h = jnp.dot(h.astype(jnp.bfloat16), wb_ref[...].astype(jnp.bfloat16),
                preferred_element_type=jnp.float32)
    h = h + bb_ref[...]
    if relu_after:
        h = jnp.maximum(h, 0.0)
        out_ref[0] = h[:, :H]
        out_ref[1] = h[:, H:]
    else:
        out_ref[...] = h


def _tc_mlp_call(relu_after, xs, agg, wa, ba, wb, bb):
    grid = N // _R
    split_spec = pl.BlockSpec((NSC, _R, H), lambda i: (0, i, 0))
    full = pl.BlockSpec((D, D), lambda i: (0, 0))
    bias = pl.BlockSpec((1, D), lambda i: (0, 0))
    if relu_after:
        out_shape = jax.ShapeDtypeStruct((NSC, N, H), jnp.float32)
        out_spec = split_spec
    else:
        out_shape = jax.ShapeDtypeStruct((N, D), jnp.float32)
        out_spec = pl.BlockSpec((_R, D), lambda i: (i, 0))
    return pl.pallas_call(
        functools.partial(_tc_body, relu_after),
        grid=(grid,),
        in_specs=[split_spec, split_spec, full, bias, full, bias],
        out_specs=out_spec,
        out_shape=out_shape,
    )(xs, agg, wa, ba, wb, bb)


def kernel(x, edge_index, W1a, b1a, W1b, b1b, W2a, b2a, W2b, b2b):
    src = edge_index[0]
    dst = edge_index[1]
    pad = E_PAD - E
    srcp = jnp.concatenate([src, jnp.zeros((pad,), jnp.int32)])
    src3 = jnp.stack([srcp, srcp + N]).reshape(NSC, NT * CHUNKS, CH)
    dst3 = jnp.concatenate([dst, jnp.full((pad,), N, jnp.int32)])
    dst3 = dst3.reshape(NT * CHUNKS, CH)
    zeros = jnp.zeros((ZROWS, H), jnp.float32)

    x2 = jnp.stack([x[:, :H], x[:, H:]])
    agg1 = _sc_agg(x2.reshape(NSC * N, H), src3, dst3, zeros)
    h2 = _tc_mlp_call(True, x2, agg1, W1a, b1a.reshape(1, D), W1b,
                      b1b.reshape(1, D))
    agg2 = _sc_agg(h2.reshape(NSC * N, H), src3, dst3, zeros)
    out = _tc_mlp_call(False, h2, agg2, W2a, b2a.reshape(1, D), W2b,
                       b2b.reshape(1, D))
    return out
